# baseline (speedup 1.0000x reference)
import functools
import jax
import jax.numpy as jnp
from jax.experimental import pallas as pl

N, E, D = 10000, 160000, 256


def _proj_body(nf_ref, w_ref, p1_ref, p2_ref):
    p = jnp.dot(nf_ref[...], w_ref[...], preferred_element_type=jnp.float32)
    p1_ref[...] = p[:, : 2 * D]
    p2_ref[...] = p[:, 2 * D :]


def _node_proj(nf, W12):
    bn = 2000
    return pl.pallas_call(
        _proj_body,
        grid=(N // bn,),
        in_specs=[
            pl.BlockSpec((bn, D), lambda i: (i, 0)),
            pl.BlockSpec((D, 4 * D), lambda i: (0, 0)),
        ],
        out_specs=[
            pl.BlockSpec((bn, 2 * D), lambda i: (i, 0)),
            pl.BlockSpec((bn, 2 * D), lambda i: (i, 0)),
        ],
        out_shape=[
            jax.ShapeDtypeStruct((N, 2 * D), jnp.float32),
            jax.ShapeDtypeStruct((N, 2 * D), jnp.float32),
        ],
    )(nf, W12)


def _edge_body(g_ref, ef_ref, w1c_ref, b1_ref, w2_ref, b2_ref, m_ref):
    h = g_ref[...] + jnp.dot(ef_ref[...], w1c_ref[...], preferred_element_type=jnp.float32)
    h = jax.nn.relu(h + b1_ref[...])
    m_ref[...] = jnp.dot(h, w2_ref[...], preferred_element_type=jnp.float32) + b2_ref[...]


def _edge_mlp(g, ef, W1c, b1, W2, b2):
    be = 2000
    return pl.pallas_call(
        _edge_body,
        grid=(E // be,),
        in_specs=[
            pl.BlockSpec((be, 2 * D), lambda i: (i, 0)),
            pl.BlockSpec((be, D), lambda i: (i, 0)),
            pl.BlockSpec((D, 2 * D), lambda i: (0, 0)),
            pl.BlockSpec((1, 2 * D), lambda i: (0, 0)),
            pl.BlockSpec((2 * D, D), lambda i: (0, 0)),
            pl.BlockSpec((1, D), lambda i: (0, 0)),
        ],
        out_specs=pl.BlockSpec((be, D), lambda i: (i, 0)),
        out_shape=jax.ShapeDtypeStruct((E, D), jnp.float32),
    )(g, ef, W1c, b1.reshape(1, -1), W2, b2.reshape(1, -1))


def _node_body(nf_ref, agg_ref, u1a_ref, u1b_ref, bu1_ref, u2_ref, bu2_ref,
               gamma_ref, beta_ref, out_ref):
    nf = nf_ref[...]
    t = (jnp.dot(nf, u1a_ref[...], preferred_element_type=jnp.float32)
         + jnp.dot(agg_ref[...], u1b_ref[...], preferred_element_type=jnp.float32)
         + bu1_ref[...])
    u = jnp.dot(jax.nn.relu(t), u2_ref[...], preferred_element_type=jnp.float32) + bu2_ref[...]
    res = nf + u
    mean = jnp.mean(res, axis=-1, keepdims=True)
    var = jnp.mean((res - mean) ** 2, axis=-1, keepdims=True)
    out_ref[...] = (res - mean) * jax.lax.rsqrt(var + 1e-5) * gamma_ref[...] + beta_ref[...]


def _node_mlp(nf, agg, U1, bu1, U2, bu2, gamma, beta):
    bn = 2000
    return pl.pallas_call(
        _node_body,
        grid=(N // bn,),
        in_specs=[
            pl.BlockSpec((bn, D), lambda i: (i, 0)),
            pl.BlockSpec((bn, D), lambda i: (i, 0)),
            pl.BlockSpec((D, D), lambda i: (0, 0)),
            pl.BlockSpec((D, D), lambda i: (0, 0)),
            pl.BlockSpec((1, D), lambda i: (0, 0)),
            pl.BlockSpec((D, D), lambda i: (0, 0)),
            pl.BlockSpec((1, D), lambda i: (0, 0)),
            pl.BlockSpec((1, D), lambda i: (0, 0)),
            pl.BlockSpec((1, D), lambda i: (0, 0)),
        ],
        out_specs=pl.BlockSpec((bn, D), lambda i: (i, 0)),
        out_shape=jax.ShapeDtypeStruct((N, D), jnp.float32),
    )(nf, agg, U1[:D], U1[D:], bu1.reshape(1, -1), U2, bu2.reshape(1, -1),
      gamma.reshape(1, -1), beta.reshape(1, -1))


def kernel(node_features, edge_features, edge_index, W1, b1, W2, b2,
           U1, bu1, U2, bu2, gamma, beta):
    src = edge_index[0]
    tgt = edge_index[1]
    Wcat = jnp.concatenate([W1[:D], W1[D : 2 * D]], axis=1)
    P1, P2 = _node_proj(node_features, Wcat)
    g = P1[src] + P2[tgt]
    m = _edge_mlp(g, edge_features, W1[2 * D :], b1, W2, b2)
    agg = jnp.zeros((N, D), jnp.float32).at[tgt].add(m)
    return _node_mlp(node_features, agg, U1, bu1, U2, bu2, gamma, beta)


# speedup vs baseline: 1.1836x; 1.1836x over previous
import functools
import jax
import jax.numpy as jnp
from jax import lax
from jax.experimental import pallas as pl
from jax.experimental.pallas import tpu as pltpu
from jax.experimental.pallas import tpu_sc as plsc

N, E, D = 10000, 160000, 256
NC, NS = 2, 16
HD = D // 2
EPT = E // NS
CH = 400
NROUND = 3
RR = 3584
NP = NROUND * RR
AR = RR + 128
ART = AR // NS
RRT = RR // NS


def _proj_body(nf_ref, w_ref, p1_ref, p2_ref):
    p = jnp.dot(nf_ref[...], w_ref[...], preferred_element_type=jnp.float32)
    p1_ref[...] = p[:, : 2 * D]
    p2_ref[...] = p[:, 2 * D :]


def _node_proj(nf, W12):
    bn = 2000
    return pl.pallas_call(
        _proj_body,
        grid=(N // bn,),
        in_specs=[
            pl.BlockSpec((bn, D), lambda i: (i, 0)),
            pl.BlockSpec((D, 4 * D), lambda i: (0, 0)),
        ],
        out_specs=[
            pl.BlockSpec((bn, 2 * D), lambda i: (i, 0)),
            pl.BlockSpec((bn, 2 * D), lambda i: (i, 0)),
        ],
        out_shape=[
            jax.ShapeDtypeStruct((N, 2 * D), jnp.float32),
            jax.ShapeDtypeStruct((N, 2 * D), jnp.float32),
        ],
    )(nf, W12)


def _edge_body(g_ref, ef_ref, w1c_ref, b1_ref, w2_ref, b2_ref, m_ref):
    h = g_ref[...] + jnp.dot(ef_ref[...], w1c_ref[...], preferred_element_type=jnp.float32)
    h = jax.nn.relu(h + b1_ref[...])
    m = jnp.dot(h, w2_ref[...], preferred_element_type=jnp.float32) + b2_ref[...]
    m_ref[0] = m[:, :HD]
    m_ref[1] = m[:, HD:]


def _edge_mlp(g, ef, W1c, b1, W2, b2):
    be = 2000
    return pl.pallas_call(
        _edge_body,
        grid=(E // be,),
        in_specs=[
            pl.BlockSpec((be, 2 * D), lambda i: (i, 0)),
            pl.BlockSpec((be, D), lambda i: (i, 0)),
            pl.BlockSpec((D, 2 * D), lambda i: (0, 0)),
            pl.BlockSpec((1, 2 * D), lambda i: (0, 0)),
            pl.BlockSpec((2 * D, D), lambda i: (0, 0)),
            pl.BlockSpec((1, D), lambda i: (0, 0)),
        ],
        out_specs=pl.BlockSpec((NC, be, HD), lambda i: (0, i, 0)),
        out_shape=jax.ShapeDtypeStruct((NC, E, HD), jnp.float32),
    )(g, ef, W1c, b1.reshape(1, -1), W2, b2.reshape(1, -1))


def _sc_scatter_body(m_hbm, tgt_hbm, zeros_hbm, out_hbm, m_v, idx_v, idx2_v, z_v, acc_sh):
    c = lax.axis_index("c")
    s = lax.axis_index("s")
    for r in range(NROUND):
        base = r * RR
        pltpu.sync_copy(zeros_hbm, z_v)
        pltpu.sync_copy(z_v, acc_sh.at[pl.ds(s * ART, ART)])
        plsc.subcore_barrier()

        def chunk(i, _):
            e0 = s * EPT + i * CH

            @pl.when(c == 0)
            def _():
                pltpu.sync_copy(m_hbm.at[0, pl.ds(e0, CH)], m_v)

            @pl.when(c == 1)
            def _():
                pltpu.sync_copy(m_hbm.at[1, pl.ds(e0, CH)], m_v)

            pltpu.sync_copy(tgt_hbm.at[pl.ds(e0, CH)], idx_v)
            for k in range(CH // 16):
                v = idx_v[pl.ds(k * 16, 16)]
                rel = v - base
                inb = (rel >= 0) & (rel < RR)
                trash = RR + (v & 127)
                idx2_v[pl.ds(k * 16, 16)] = jnp.where(inb, rel, trash)
            pltpu.sync_copy(m_v, acc_sh.at[idx2_v], add=True)
            return 0

        lax.fori_loop(0, EPT // CH, chunk, 0)
        plsc.subcore_barrier()
        pltpu.sync_copy(acc_sh.at[pl.ds(s * RRT, RRT)], z_v.at[pl.ds(0, RRT)])

        @pl.when(c == 0)
        def _():
            pltpu.sync_copy(z_v.at[pl.ds(0, RRT)],
                            out_hbm.at[0, pl.ds(base + s * RRT, RRT)])

        @pl.when(c == 1)
        def _():
            pltpu.sync_copy(z_v.at[pl.ds(0, RRT)],
                            out_hbm.at[1, pl.ds(base + s * RRT, RRT)])

        plsc.subcore_barrier()


def _sc_scatter(m, tgt, zeros_art):
    mesh = plsc.VectorSubcoreMesh(
        core_axis_name="c", subcore_axis_name="s", num_cores=NC, num_subcores=NS
    )
    return pl.kernel(
        _sc_scatter_body,
        out_type=jax.ShapeDtypeStruct((NC, NP, HD), jnp.float32),
        mesh=mesh,
        scratch_types=[
            pltpu.VMEM((CH, HD), jnp.float32),
            pltpu.VMEM((CH,), jnp.int32),
            pltpu.VMEM((CH,), jnp.int32),
            pltpu.VMEM((ART, HD), jnp.float32),
            pltpu.VMEM_SHARED((AR, HD), jnp.float32),
        ],
    )(m, tgt, zeros_art)


def _node_body(nf_ref, agg_ref, u1a_ref, u1b0_ref, u1b1_ref, bu1_ref, u2_ref,
               bu2_ref, gamma_ref, beta_ref, out_ref):
    nf = nf_ref[...]
    t = (jnp.dot(nf, u1a_ref[...], preferred_element_type=jnp.float32)
         + jnp.dot(agg_ref[0], u1b0_ref[...], preferred_element_type=jnp.float32)
         + jnp.dot(agg_ref[1], u1b1_ref[...], preferred_element_type=jnp.float32)
         + bu1_ref[...])
    u = jnp.dot(jax.nn.relu(t), u2_ref[...], preferred_element_type=jnp.float32) + bu2_ref[...]
    res = nf + u
    mean = jnp.mean(res, axis=-1, keepdims=True)
    var = jnp.mean((res - mean) ** 2, axis=-1, keepdims=True)
    out_ref[...] = (res - mean) * jax.lax.rsqrt(var + 1e-5) * gamma_ref[...] + beta_ref[...]


def _node_mlp(nf, agg, U1, bu1, U2, bu2, gamma, beta):
    bn = 2000
    return pl.pallas_call(
        _node_body,
        grid=(N // bn,),
        in_specs=[
            pl.BlockSpec((bn, D), lambda i: (i, 0)),
            pl.BlockSpec((NC, bn, HD), lambda i: (0, i, 0)),
            pl.BlockSpec((D, D), lambda i: (0, 0)),
            pl.BlockSpec((HD, D), lambda i: (0, 0)),
            pl.BlockSpec((HD, D), lambda i: (0, 0)),
            pl.BlockSpec((1, D), lambda i: (0, 0)),
            pl.BlockSpec((D, D), lambda i: (0, 0)),
            pl.BlockSpec((1, D), lambda i: (0, 0)),
            pl.BlockSpec((1, D), lambda i: (0, 0)),
            pl.BlockSpec((1, D), lambda i: (0, 0)),
        ],
        out_specs=pl.BlockSpec((bn, D), lambda i: (i, 0)),
        out_shape=jax.ShapeDtypeStruct((N, D), jnp.float32),
    )(nf, agg, U1[:D], U1[D : D + HD], U1[D + HD :], bu1.reshape(1, -1), U2,
      bu2.reshape(1, -1), gamma.reshape(1, -1), beta.reshape(1, -1))


def kernel(node_features, edge_features, edge_index, W1, b1, W2, b2,
           U1, bu1, U2, bu2, gamma, beta):
    src = edge_index[0]
    tgt = edge_index[1]
    Wcat = jnp.concatenate([W1[:D], W1[D : 2 * D]], axis=1)
    P1, P2 = _node_proj(node_features, Wcat)
    g = P1[src] + P2[tgt]
    m = _edge_mlp(g, edge_features, W1[2 * D :], b1, W2, b2)
    zeros_art = jnp.zeros((ART, HD), jnp.float32)
    agg = _sc_scatter(m, tgt, zeros_art)
    return _node_mlp(node_features, agg, U1, bu1, U2, bu2, gamma, beta)


# speedup vs baseline: 2.3619x; 1.9956x over previous
import functools
import jax
import jax.numpy as jnp
from jax import lax
from jax.experimental import pallas as pl
from jax.experimental.pallas import tpu as pltpu
from jax.experimental.pallas import tpu_sc as plsc

N, E, D = 10000, 160000, 256
NC, NS = 2, 16
HD = D // 2
EPT = E // NS
CH = 400
NROUND = 3
RR = 3584
NP = NROUND * RR
AR = RR + 128
ART = AR // NS
RRT = RR // NS
NW = NC * NS
EPW = E // NW
CHG = 200


def _pack_bf16_pair(hi, lo):
    hb = jax.lax.bitcast_convert_type(hi.astype(jnp.bfloat16).astype(jnp.float32), jnp.int32)
    lb = jax.lax.bitcast_convert_type(lo.astype(jnp.bfloat16).astype(jnp.float32), jnp.int32)
    return hb | jax.lax.shift_right_logical(lb, 16)


def _unpack_hi(w):
    return jax.lax.bitcast_convert_type(w & jnp.int32(-65536), jnp.float32)


def _unpack_lo(w):
    return jax.lax.bitcast_convert_type(jax.lax.shift_left(w, 16), jnp.float32)


def _proj_body(nf_ref, w_ref, p1_ref, p2_ref):
    p = jnp.dot(nf_ref[...], w_ref[...], preferred_element_type=jnp.float32)
    p1 = p[:, : 2 * D]
    p2 = p[:, 2 * D :]
    p1_ref[...] = _pack_bf16_pair(p1[:, :D], p1[:, D:])
    p2_ref[...] = _pack_bf16_pair(p2[:, :D], p2[:, D:])


def _node_proj(nf, W12):
    bn = 2000
    return pl.pallas_call(
        _proj_body,
        grid=(N // bn,),
        in_specs=[
            pl.BlockSpec((bn, D), lambda i: (i, 0)),
            pl.BlockSpec((D, 4 * D), lambda i: (0, 0)),
        ],
        out_specs=[
            pl.BlockSpec((bn, D), lambda i: (i, 0)),
            pl.BlockSpec((bn, D), lambda i: (i, 0)),
        ],
        out_shape=[
            jax.ShapeDtypeStruct((N, D), jnp.int32),
            jax.ShapeDtypeStruct((N, D), jnp.int32),
        ],
    )(nf, W12)


def _sc_gather_body(p1_hbm, p2_hbm, src_hbm, tgt_hbm, g1_hbm, g2_hbm,
                    idx1_v, idx2_v, rows1_v, rows2_v, sem1, sem2):
    c = lax.axis_index("c")
    s = lax.axis_index("s")
    w = s * NC + c

    def chunk(i, _):
        e0 = w * EPW + i * CHG
        pltpu.sync_copy(src_hbm.at[pl.ds(e0, CHG)], idx1_v)
        pltpu.sync_copy(tgt_hbm.at[pl.ds(e0, CHG)], idx2_v)
        cp1 = pltpu.async_copy(p1_hbm.at[idx1_v], rows1_v, sem1)
        cp2 = pltpu.async_copy(p2_hbm.at[idx2_v], rows2_v, sem2)
        cp1.wait()
        cp2.wait()
        pltpu.sync_copy(rows1_v, g1_hbm.at[pl.ds(e0, CHG)])
        pltpu.sync_copy(rows2_v, g2_hbm.at[pl.ds(e0, CHG)])
        return 0

    lax.fori_loop(0, EPW // CHG, chunk, 0)


def _sc_gather(P1, P2, src, tgt):
    mesh = plsc.VectorSubcoreMesh(
        core_axis_name="c", subcore_axis_name="s", num_cores=NC, num_subcores=NS
    )
    return pl.kernel(
        _sc_gather_body,
        out_type=[
            jax.ShapeDtypeStruct((E, D), jnp.int32),
            jax.ShapeDtypeStruct((E, D), jnp.int32),
        ],
        mesh=mesh,
        scratch_types=[
            pltpu.VMEM((CHG,), jnp.int32),
            pltpu.VMEM((CHG,), jnp.int32),
            pltpu.VMEM((CHG, D), jnp.int32),
            pltpu.VMEM((CHG, D), jnp.int32),
            pltpu.SemaphoreType.DMA,
            pltpu.SemaphoreType.DMA,
        ],
    )(P1, P2, src, tgt)


def _edge_body(g1_ref, g2_ref, ef_ref, w1c_ref, b1_ref, w2a_ref, w2b_ref,
               b2_ref, m_ref):
    g1w = g1_ref[...]
    g2w = g2_ref[...]
    ef = ef_ref[...].astype(jnp.bfloat16)
    t = jnp.dot(ef, w1c_ref[...], preferred_element_type=jnp.float32) + b1_ref[...]
    h1 = jax.nn.relu(_unpack_hi(g1w) + _unpack_hi(g2w) + t[:, :D]).astype(jnp.bfloat16)
    h2 = jax.nn.relu(_unpack_lo(g1w) + _unpack_lo(g2w) + t[:, D:]).astype(jnp.bfloat16)
    m = (jnp.dot(h1, w2a_ref[...], preferred_element_type=jnp.float32)
         + jnp.dot(h2, w2b_ref[...], preferred_element_type=jnp.float32)
         + b2_ref[...])
    m_ref[0] = m[:, :HD]
    m_ref[1] = m[:, HD:]


def _edge_mlp(g1, g2, ef, W1c, b1, W2, b2):
    be = 2000
    return pl.pallas_call(
        _edge_body,
        grid=(E // be,),
        in_specs=[
            pl.BlockSpec((be, D), lambda i: (i, 0)),
            pl.BlockSpec((be, D), lambda i: (i, 0)),
            pl.BlockSpec((be, D), lambda i: (i, 0)),
            pl.BlockSpec((D, 2 * D), lambda i: (0, 0)),
            pl.BlockSpec((1, 2 * D), lambda i: (0, 0)),
            pl.BlockSpec((D, D), lambda i: (0, 0)),
            pl.BlockSpec((D, D), lambda i: (0, 0)),
            pl.BlockSpec((1, D), lambda i: (0, 0)),
        ],
        out_specs=pl.BlockSpec((NC, be, HD), lambda i: (0, i, 0)),
        out_shape=jax.ShapeDtypeStruct((NC, E, HD), jnp.float32),
    )(g1, g2, ef, W1c.astype(jnp.bfloat16), b1.reshape(1, -1),
      W2[:D].astype(jnp.bfloat16), W2[D:].astype(jnp.bfloat16), b2.reshape(1, -1))


def _sc_scatter_body(m_hbm, tgt_hbm, zeros_hbm, out_hbm, m_v, idx_v, idx2_v, z_v, acc_sh):
    c = lax.axis_index("c")
    s = lax.axis_index("s")
    for r in range(NROUND):
        base = r * RR
        pltpu.sync_copy(zeros_hbm, z_v)
        pltpu.sync_copy(z_v, acc_sh.at[pl.ds(s * ART, ART)])
        plsc.subcore_barrier()

        def chunk(i, _):
            e0 = s * EPT + i * CH

            @pl.when(c == 0)
            def _():
                pltpu.sync_copy(m_hbm.at[0, pl.ds(e0, CH)], m_v)

            @pl.when(c == 1)
            def _():
                pltpu.sync_copy(m_hbm.at[1, pl.ds(e0, CH)], m_v)

            pltpu.sync_copy(tgt_hbm.at[pl.ds(e0, CH)], idx_v)
            for k in range(CH // 16):
                v = idx_v[pl.ds(k * 16, 16)]
                rel = v - base
                inb = (rel >= 0) & (rel < RR)
                trash = RR + (v & 127)
                idx2_v[pl.ds(k * 16, 16)] = jnp.where(inb, rel, trash)
            pltpu.sync_copy(m_v, acc_sh.at[idx2_v], add=True)
            return 0

        lax.fori_loop(0, EPT // CH, chunk, 0)
        plsc.subcore_barrier()
        pltpu.sync_copy(acc_sh.at[pl.ds(s * RRT, RRT)], z_v.at[pl.ds(0, RRT)])

        @pl.when(c == 0)
        def _():
            pltpu.sync_copy(z_v.at[pl.ds(0, RRT)],
                            out_hbm.at[0, pl.ds(base + s * RRT, RRT)])

        @pl.when(c == 1)
        def _():
            pltpu.sync_copy(z_v.at[pl.ds(0, RRT)],
                            out_hbm.at[1, pl.ds(base + s * RRT, RRT)])

        plsc.subcore_barrier()


def _sc_scatter(m, tgt, zeros_art):
    mesh = plsc.VectorSubcoreMesh(
        core_axis_name="c", subcore_axis_name="s", num_cores=NC, num_subcores=NS
    )
    return pl.kernel(
        _sc_scatter_body,
        out_type=jax.ShapeDtypeStruct((NC, NP, HD), jnp.float32),
        mesh=mesh,
        scratch_types=[
            pltpu.VMEM((CH, HD), jnp.float32),
            pltpu.VMEM((CH,), jnp.int32),
            pltpu.VMEM((CH,), jnp.int32),
            pltpu.VMEM((ART, HD), jnp.float32),
            pltpu.VMEM_SHARED((AR, HD), jnp.float32),
        ],
    )(m, tgt, zeros_art)


def _node_body(nf_ref, agg_ref, u1a_ref, u1b0_ref, u1b1_ref, bu1_ref, u2_ref,
               bu2_ref, gamma_ref, beta_ref, out_ref):
    nf = nf_ref[...]
    t = (jnp.dot(nf, u1a_ref[...], preferred_element_type=jnp.float32)
         + jnp.dot(agg_ref[0], u1b0_ref[...], preferred_element_type=jnp.float32)
         + jnp.dot(agg_ref[1], u1b1_ref[...], preferred_element_type=jnp.float32)
         + bu1_ref[...])
    u = jnp.dot(jax.nn.relu(t), u2_ref[...], preferred_element_type=jnp.float32) + bu2_ref[...]
    res = nf + u
    mean = jnp.mean(res, axis=-1, keepdims=True)
    var = jnp.mean((res - mean) ** 2, axis=-1, keepdims=True)
    out_ref[...] = (res - mean) * jax.lax.rsqrt(var + 1e-5) * gamma_ref[...] + beta_ref[...]


def _node_mlp(nf, agg, U1, bu1, U2, bu2, gamma, beta):
    bn = 2000
    return pl.pallas_call(
        _node_body,
        grid=(N // bn,),
        in_specs=[
            pl.BlockSpec((bn, D), lambda i: (i, 0)),
            pl.BlockSpec((NC, bn, HD), lambda i: (0, i, 0)),
            pl.BlockSpec((D, D), lambda i: (0, 0)),
            pl.BlockSpec((HD, D), lambda i: (0, 0)),
            pl.BlockSpec((HD, D), lambda i: (0, 0)),
            pl.BlockSpec((1, D), lambda i: (0, 0)),
            pl.BlockSpec((D, D), lambda i: (0, 0)),
            pl.BlockSpec((1, D), lambda i: (0, 0)),
            pl.BlockSpec((1, D), lambda i: (0, 0)),
            pl.BlockSpec((1, D), lambda i: (0, 0)),
        ],
        out_specs=pl.BlockSpec((bn, D), lambda i: (i, 0)),
        out_shape=jax.ShapeDtypeStruct((N, D), jnp.float32),
    )(nf, agg, U1[:D], U1[D : D + HD], U1[D + HD :], bu1.reshape(1, -1), U2,
      bu2.reshape(1, -1), gamma.reshape(1, -1), beta.reshape(1, -1))


def kernel(node_features, edge_features, edge_index, W1, b1, W2, b2,
           U1, bu1, U2, bu2, gamma, beta):
    src = edge_index[0]
    tgt = edge_index[1]
    Wcat = jnp.concatenate([W1[:D], W1[D : 2 * D]], axis=1)
    P1, P2 = _node_proj(node_features, Wcat)
    g1, g2 = _sc_gather(P1, P2, src, tgt)
    m = _edge_mlp(g1, g2, edge_features, W1[2 * D :], b1, W2, b2)
    zeros_art = jnp.zeros((ART, HD), jnp.float32)
    agg = _sc_scatter(m, tgt, zeros_art)
    return _node_mlp(node_features, agg, U1, bu1, U2, bu2, gamma, beta)


# speedup vs baseline: 2.4575x; 1.0405x over previous
import functools
import jax
import jax.numpy as jnp
from jax import lax
from jax.experimental import pallas as pl
from jax.experimental.pallas import tpu as pltpu
from jax.experimental.pallas import tpu_sc as plsc

N, E, D = 10000, 160000, 256
NC, NS = 2, 16
HD = D // 2
EPT = E // NS
CH = 400
NROUND = 3
RR = 3584
NP = NROUND * RR
AR = RR + 128
ART = AR // NS
RRT = RR // NS
EPTG = E // NS
CHG = 200


def _pack_bf16_pair(hi, lo):
    hb = jax.lax.bitcast_convert_type(hi.astype(jnp.bfloat16).astype(jnp.float32), jnp.int32)
    lb = jax.lax.bitcast_convert_type(lo.astype(jnp.bfloat16).astype(jnp.float32), jnp.int32)
    return hb | jax.lax.shift_right_logical(lb, 16)


def _unpack_hi(w):
    return jax.lax.bitcast_convert_type(w & jnp.int32(-65536), jnp.float32)


def _unpack_lo(w):
    return jax.lax.bitcast_convert_type(jax.lax.shift_left(w, 16), jnp.float32)


def _proj_body(nf_ref, w_ref, p1_ref, p2_ref):
    p = jnp.dot(nf_ref[...], w_ref[...], preferred_element_type=jnp.float32)
    p1 = p[:, : 2 * D]
    p2 = p[:, 2 * D :]
    p1_ref[...] = _pack_bf16_pair(p1[:, :D], p1[:, D:])
    p2_ref[...] = _pack_bf16_pair(p2[:, :D], p2[:, D:])


def _node_proj(nf, W12):
    bn = 2000
    return pl.pallas_call(
        _proj_body,
        grid=(N // bn,),
        in_specs=[
            pl.BlockSpec((bn, D), lambda i: (i, 0)),
            pl.BlockSpec((D, 4 * D), lambda i: (0, 0)),
        ],
        out_specs=[
            pl.BlockSpec((bn, D), lambda i: (i, 0)),
            pl.BlockSpec((bn, D), lambda i: (i, 0)),
        ],
        out_shape=[
            jax.ShapeDtypeStruct((N, D), jnp.int32),
            jax.ShapeDtypeStruct((N, D), jnp.int32),
        ],
    )(nf, W12)


def _gather_endpoint(tab, idx_hbm, out, idx_all, ra, rb, sa, sb, e_base):
    pltpu.sync_copy(idx_hbm.at[pl.ds(e_base, EPTG)], idx_all)
    nch = EPTG // CHG

    def start(i, buf, sem):
        pltpu.async_copy(tab.at[idx_all.at[pl.ds(i * CHG, CHG)]], buf, sem)

    def wait(buf, sem):
        pltpu.make_async_copy(tab.at[pl.ds(0, CHG)], buf, sem).wait()

    start(0, ra, sa)

    def body(j, _):
        i0 = 2 * j
        start(i0 + 1, rb, sb)
        wait(ra, sa)
        pltpu.sync_copy(ra, out.at[pl.ds(e_base + i0 * CHG, CHG)])

        @pl.when(j < nch // 2 - 1)
        def _():
            start(i0 + 2, ra, sa)

        wait(rb, sb)
        pltpu.sync_copy(rb, out.at[pl.ds(e_base + (i0 + 1) * CHG, CHG)])
        return 0

    lax.fori_loop(0, nch // 2, body, 0)


def _sc_gather_body(p1_hbm, p2_hbm, src_hbm, tgt_hbm, g1_hbm, g2_hbm,
                    idx_all, rows_a, rows_b, sem_a, sem_b):
    c = lax.axis_index("c")
    s = lax.axis_index("s")
    e_base = s * EPTG

    @pl.when(c == 0)
    def _():
        _gather_endpoint(p1_hbm, src_hbm, g1_hbm, idx_all, rows_a, rows_b,
                         sem_a, sem_b, e_base)

    @pl.when(c == 1)
    def _():
        _gather_endpoint(p2_hbm, tgt_hbm, g2_hbm, idx_all, rows_a, rows_b,
                         sem_a, sem_b, e_base)


def _sc_gather(P1, P2, src, tgt):
    mesh = plsc.VectorSubcoreMesh(
        core_axis_name="c", subcore_axis_name="s", num_cores=NC, num_subcores=NS
    )
    return pl.kernel(
        _sc_gather_body,
        out_type=[
            jax.ShapeDtypeStruct((E, D), jnp.int32),
            jax.ShapeDtypeStruct((E, D), jnp.int32),
        ],
        mesh=mesh,
        scratch_types=[
            pltpu.VMEM((EPTG,), jnp.int32),
            pltpu.VMEM((CHG, D), jnp.int32),
            pltpu.VMEM((CHG, D), jnp.int32),
            pltpu.SemaphoreType.DMA,
            pltpu.SemaphoreType.DMA,
        ],
    )(P1, P2, src, tgt)


def _edge_body(g1_ref, g2_ref, ef_ref, w1c_ref, b1_ref, w2a_ref, w2b_ref,
               b2_ref, m_ref):
    g1w = g1_ref[...]
    g2w = g2_ref[...]
    ef = ef_ref[...].astype(jnp.bfloat16)
    t = jnp.dot(ef, w1c_ref[...], preferred_element_type=jnp.float32) + b1_ref[...]
    h1 = jax.nn.relu(_unpack_hi(g1w) + _unpack_hi(g2w) + t[:, :D]).astype(jnp.bfloat16)
    h2 = jax.nn.relu(_unpack_lo(g1w) + _unpack_lo(g2w) + t[:, D:]).astype(jnp.bfloat16)
    m = (jnp.dot(h1, w2a_ref[...], preferred_element_type=jnp.float32)
         + jnp.dot(h2, w2b_ref[...], preferred_element_type=jnp.float32)
         + b2_ref[...])
    m_ref[0] = m[:, :HD]
    m_ref[1] = m[:, HD:]


def _edge_mlp(g1, g2, ef, W1c, b1, W2, b2):
    be = 2000
    return pl.pallas_call(
        _edge_body,
        grid=(E // be,),
        in_specs=[
            pl.BlockSpec((be, D), lambda i: (i, 0)),
            pl.BlockSpec((be, D), lambda i: (i, 0)),
            pl.BlockSpec((be, D), lambda i: (i, 0)),
            pl.BlockSpec((D, 2 * D), lambda i: (0, 0)),
            pl.BlockSpec((1, 2 * D), lambda i: (0, 0)),
            pl.BlockSpec((D, D), lambda i: (0, 0)),
            pl.BlockSpec((D, D), lambda i: (0, 0)),
            pl.BlockSpec((1, D), lambda i: (0, 0)),
        ],
        out_specs=pl.BlockSpec((NC, be, HD), lambda i: (0, i, 0)),
        out_shape=jax.ShapeDtypeStruct((NC, E, HD), jnp.float32),
    )(g1, g2, ef, W1c.astype(jnp.bfloat16), b1.reshape(1, -1),
      W2[:D].astype(jnp.bfloat16), W2[D:].astype(jnp.bfloat16), b2.reshape(1, -1))


def _sc_scatter_body(m_hbm, tgt_hbm, zeros_hbm, out_hbm, m_v, idx_v, idx2_v, z_v, acc_sh):
    c = lax.axis_index("c")
    s = lax.axis_index("s")
    for r in range(NROUND):
        base = r * RR
        pltpu.sync_copy(zeros_hbm, z_v)
        pltpu.sync_copy(z_v, acc_sh.at[pl.ds(s * ART, ART)])
        plsc.subcore_barrier()

        def chunk(i, _):
            e0 = s * EPT + i * CH

            @pl.when(c == 0)
            def _():
                pltpu.sync_copy(m_hbm.at[0, pl.ds(e0, CH)], m_v)

            @pl.when(c == 1)
            def _():
                pltpu.sync_copy(m_hbm.at[1, pl.ds(e0, CH)], m_v)

            pltpu.sync_copy(tgt_hbm.at[pl.ds(e0, CH)], idx_v)
            for k in range(CH // 16):
                v = idx_v[pl.ds(k * 16, 16)]
                rel = v - base
                inb = (rel >= 0) & (rel < RR)
                trash = RR + (v & 127)
                idx2_v[pl.ds(k * 16, 16)] = jnp.where(inb, rel, trash)
            pltpu.sync_copy(m_v, acc_sh.at[idx2_v], add=True)
            return 0

        lax.fori_loop(0, EPT // CH, chunk, 0)
        plsc.subcore_barrier()
        pltpu.sync_copy(acc_sh.at[pl.ds(s * RRT, RRT)], z_v.at[pl.ds(0, RRT)])

        @pl.when(c == 0)
        def _():
            pltpu.sync_copy(z_v.at[pl.ds(0, RRT)],
                            out_hbm.at[0, pl.ds(base + s * RRT, RRT)])

        @pl.when(c == 1)
        def _():
            pltpu.sync_copy(z_v.at[pl.ds(0, RRT)],
                            out_hbm.at[1, pl.ds(base + s * RRT, RRT)])

        plsc.subcore_barrier()


def _sc_scatter(m, tgt, zeros_art):
    mesh = plsc.VectorSubcoreMesh(
        core_axis_name="c", subcore_axis_name="s", num_cores=NC, num_subcores=NS
    )
    return pl.kernel(
        _sc_scatter_body,
        out_type=jax.ShapeDtypeStruct((NC, NP, HD), jnp.float32),
        mesh=mesh,
        scratch_types=[
            pltpu.VMEM((CH, HD), jnp.float32),
            pltpu.VMEM((CH,), jnp.int32),
            pltpu.VMEM((CH,), jnp.int32),
            pltpu.VMEM((ART, HD), jnp.float32),
            pltpu.VMEM_SHARED((AR, HD), jnp.float32),
        ],
    )(m, tgt, zeros_art)


def _node_body(nf_ref, agg_ref, u1a_ref, u1b0_ref, u1b1_ref, bu1_ref, u2_ref,
               bu2_ref, gamma_ref, beta_ref, out_ref):
    nf = nf_ref[...]
    t = (jnp.dot(nf, u1a_ref[...], preferred_element_type=jnp.float32)
         + jnp.dot(agg_ref[0], u1b0_ref[...], preferred_element_type=jnp.float32)
         + jnp.dot(agg_ref[1], u1b1_ref[...], preferred_element_type=jnp.float32)
         + bu1_ref[...])
    u = jnp.dot(jax.nn.relu(t), u2_ref[...], preferred_element_type=jnp.float32) + bu2_ref[...]
    res = nf + u
    mean = jnp.mean(res, axis=-1, keepdims=True)
    var = jnp.mean((res - mean) ** 2, axis=-1, keepdims=True)
    out_ref[...] = (res - mean) * jax.lax.rsqrt(var + 1e-5) * gamma_ref[...] + beta_ref[...]


def _node_mlp(nf, agg, U1, bu1, U2, bu2, gamma, beta):
    bn = 2000
    return pl.pallas_call(
        _node_body,
        grid=(N // bn,),
        in_specs=[
            pl.BlockSpec((bn, D), lambda i: (i, 0)),
            pl.BlockSpec((NC, bn, HD), lambda i: (0, i, 0)),
            pl.BlockSpec((D, D), lambda i: (0, 0)),
            pl.BlockSpec((HD, D), lambda i: (0, 0)),
            pl.BlockSpec((HD, D), lambda i: (0, 0)),
            pl.BlockSpec((1, D), lambda i: (0, 0)),
            pl.BlockSpec((D, D), lambda i: (0, 0)),
            pl.BlockSpec((1, D), lambda i: (0, 0)),
            pl.BlockSpec((1, D), lambda i: (0, 0)),
            pl.BlockSpec((1, D), lambda i: (0, 0)),
        ],
        out_specs=pl.BlockSpec((bn, D), lambda i: (i, 0)),
        out_shape=jax.ShapeDtypeStruct((N, D), jnp.float32),
    )(nf, agg, U1[:D], U1[D : D + HD], U1[D + HD :], bu1.reshape(1, -1), U2,
      bu2.reshape(1, -1), gamma.reshape(1, -1), beta.reshape(1, -1))


def kernel(node_features, edge_features, edge_index, W1, b1, W2, b2,
           U1, bu1, U2, bu2, gamma, beta):
    src = edge_index[0]
    tgt = edge_index[1]
    Wcat = jnp.concatenate([W1[:D], W1[D : 2 * D]], axis=1)
    P1, P2 = _node_proj(node_features, Wcat)
    g1, g2 = _sc_gather(P1, P2, src, tgt)
    m = _edge_mlp(g1, g2, edge_features, W1[2 * D :], b1, W2, b2)
    zeros_art = jnp.zeros((ART, HD), jnp.float32)
    agg = _sc_scatter(m, tgt, zeros_art)
    return _node_mlp(node_features, agg, U1, bu1, U2, bu2, gamma, beta)


# speedup vs baseline: 2.6118x; 1.0628x over previous
import functools
import jax
import jax.numpy as jnp
from jax import lax
from jax.experimental import pallas as pl
from jax.experimental.pallas import tpu as pltpu
from jax.experimental.pallas import tpu_sc as plsc

N, E, D = 10000, 160000, 256
NC, NS = 2, 16
HD = D // 2
CH = 400
CHG = 200
NROUND = 3
RR = 3584
NP = NROUND * RR
AR = RR + 128
ART = AR // NS
RRT = RR // NS
EA, EB = 76800, 83200
BE = 1600


def _pack_bf16_pair(hi, lo):
    hb = jax.lax.bitcast_convert_type(hi.astype(jnp.bfloat16).astype(jnp.float32), jnp.int32)
    lb = jax.lax.bitcast_convert_type(lo.astype(jnp.bfloat16).astype(jnp.float32), jnp.int32)
    return hb | jax.lax.shift_right_logical(lb, 16)


def _unpack_hi(w):
    return jax.lax.bitcast_convert_type(w & jnp.int32(-65536), jnp.float32)


def _unpack_lo(w):
    return jax.lax.bitcast_convert_type(jax.lax.shift_left(w, 16), jnp.float32)


def _proj_body(nf_ref, w_ref, p1_ref, p2_ref):
    p = jnp.dot(nf_ref[...], w_ref[...], preferred_element_type=jnp.float32)
    p1 = p[:, : 2 * D]
    p2 = p[:, 2 * D :]
    p1_ref[...] = _pack_bf16_pair(p1[:, :D], p1[:, D:])
    p2_ref[...] = _pack_bf16_pair(p2[:, :D], p2[:, D:])


def _node_proj(nf, W12):
    bn = 2000
    return pl.pallas_call(
        _proj_body,
        grid=(N // bn,),
        in_specs=[
            pl.BlockSpec((bn, D), lambda i: (i, 0)),
            pl.BlockSpec((D, 4 * D), lambda i: (0, 0)),
        ],
        out_specs=[
            pl.BlockSpec((bn, D), lambda i: (i, 0)),
            pl.BlockSpec((bn, D), lambda i: (i, 0)),
        ],
        out_shape=[
            jax.ShapeDtypeStruct((N, D), jnp.int32),
            jax.ShapeDtypeStruct((N, D), jnp.int32),
        ],
    )(nf, W12)


def _make_sc_gather(e_off, e_len):
    eptg = e_len // NS
    nch = eptg // CHG

    def endpoint(tab, idx_hbm, out, idx_all, ra, rb, sa, sb, s):
        lb = s * eptg
        pltpu.sync_copy(idx_hbm.at[pl.ds(e_off + lb, eptg)], idx_all)

        def start(i, buf, sem):
            pltpu.async_copy(tab.at[idx_all.at[pl.ds(i * CHG, CHG)]], buf, sem)

        def wait(buf, sem):
            pltpu.make_async_copy(tab.at[pl.ds(0, CHG)], buf, sem).wait()

        start(0, ra, sa)

        def body(j, _):
            i0 = 2 * j
            start(i0 + 1, rb, sb)
            wait(ra, sa)
            pltpu.sync_copy(ra, out.at[pl.ds(lb + i0 * CHG, CHG)])

            @pl.when(j < nch // 2 - 1)
            def _():
                start(i0 + 2, ra, sa)

            wait(rb, sb)
            pltpu.sync_copy(rb, out.at[pl.ds(lb + (i0 + 1) * CHG, CHG)])
            return 0

        lax.fori_loop(0, nch // 2, body, 0)

    def body(p1_hbm, p2_hbm, src_hbm, tgt_hbm, g1_hbm, g2_hbm,
             idx_all, rows_a, rows_b, sem_a, sem_b):
        c = lax.axis_index("c")
        s = lax.axis_index("s")

        @pl.when(c == 0)
        def _():
            endpoint(p1_hbm, src_hbm, g1_hbm, idx_all, rows_a, rows_b,
                     sem_a, sem_b, s)

        @pl.when(c == 1)
        def _():
            endpoint(p2_hbm, tgt_hbm, g2_hbm, idx_all, rows_a, rows_b,
                     sem_a, sem_b, s)

    mesh = plsc.VectorSubcoreMesh(
        core_axis_name="c", subcore_axis_name="s", num_cores=NC, num_subcores=NS
    )
    return pl.kernel(
        body,
        out_type=[
            jax.ShapeDtypeStruct((e_len, D), jnp.int32),
            jax.ShapeDtypeStruct((e_len, D), jnp.int32),
        ],
        mesh=mesh,
        scratch_types=[
            pltpu.VMEM((eptg,), jnp.int32),
            pltpu.VMEM((CHG, D), jnp.int32),
            pltpu.VMEM((CHG, D), jnp.int32),
            pltpu.SemaphoreType.DMA,
            pltpu.SemaphoreType.DMA,
        ],
    )


def _edge_body(g1_ref, g2_ref, ef_ref, w1c_ref, b1_ref, w2a_ref, w2b_ref,
               b2_ref, m_ref):
    g1w = g1_ref[...]
    g2w = g2_ref[...]
    ef = ef_ref[...].astype(jnp.bfloat16)
    t = jnp.dot(ef, w1c_ref[...], preferred_element_type=jnp.float32) + b1_ref[...]
    h1 = jax.nn.relu(_unpack_hi(g1w) + _unpack_hi(g2w) + t[:, :D]).astype(jnp.bfloat16)
    h2 = jax.nn.relu(_unpack_lo(g1w) + _unpack_lo(g2w) + t[:, D:]).astype(jnp.bfloat16)
    m = (jnp.dot(h1, w2a_ref[...], preferred_element_type=jnp.float32)
         + jnp.dot(h2, w2b_ref[...], preferred_element_type=jnp.float32)
         + b2_ref[...])
    m_ref[0] = m[:, :HD]
    m_ref[1] = m[:, HD:]


def _edge_mlp(e_off, e_len, g1, g2, ef, W1c, b1r, W2a, W2b, b2r):
    ob = e_off // BE
    return pl.pallas_call(
        _edge_body,
        grid=(e_len // BE,),
        in_specs=[
            pl.BlockSpec((BE, D), lambda i: (i, 0)),
            pl.BlockSpec((BE, D), lambda i: (i, 0)),
            pl.BlockSpec((BE, D), lambda i: (i + ob, 0)),
            pl.BlockSpec((D, 2 * D), lambda i: (0, 0)),
            pl.BlockSpec((1, 2 * D), lambda i: (0, 0)),
            pl.BlockSpec((D, D), lambda i: (0, 0)),
            pl.BlockSpec((D, D), lambda i: (0, 0)),
            pl.BlockSpec((1, D), lambda i: (0, 0)),
        ],
        out_specs=pl.BlockSpec((NC, BE, HD), lambda i: (0, i, 0)),
        out_shape=jax.ShapeDtypeStruct((NC, e_len, HD), jnp.float32),
    )(g1, g2, ef, W1c, b1r, W2a, W2b, b2r)


def _make_sc_scatter(e_off, e_len):
    ept = e_len // NS
    nch = ept // CH

    def body(m_hbm, tgt_hbm, zeros_hbm, out_hbm, m_v, idx_v, idx2_v, z_v, acc_sh):
        c = lax.axis_index("c")
        s = lax.axis_index("s")
        for r in range(NROUND):
            base = r * RR
            pltpu.sync_copy(zeros_hbm, z_v)
            pltpu.sync_copy(z_v, acc_sh.at[pl.ds(s * ART, ART)])
            plsc.subcore_barrier()

            def chunk(i, _):
                lb = s * ept + i * CH

                @pl.when(c == 0)
                def _():
                    pltpu.sync_copy(m_hbm.at[0, pl.ds(lb, CH)], m_v)

                @pl.when(c == 1)
                def _():
                    pltpu.sync_copy(m_hbm.at[1, pl.ds(lb, CH)], m_v)

                pltpu.sync_copy(tgt_hbm.at[pl.ds(e_off + lb, CH)], idx_v)
                for k in range(CH // 16):
                    v = idx_v[pl.ds(k * 16, 16)]
                    rel = v - base
                    inb = (rel >= 0) & (rel < RR)
                    trash = RR + (v & 127)
                    idx2_v[pl.ds(k * 16, 16)] = jnp.where(inb, rel, trash)
                pltpu.sync_copy(m_v, acc_sh.at[idx2_v], add=True)
                return 0

            lax.fori_loop(0, nch, chunk, 0)
            plsc.subcore_barrier()
            pltpu.sync_copy(acc_sh.at[pl.ds(s * RRT, RRT)], z_v.at[pl.ds(0, RRT)])

            @pl.when(c == 0)
            def _():
                pltpu.sync_copy(z_v.at[pl.ds(0, RRT)],
                                out_hbm.at[0, pl.ds(base + s * RRT, RRT)])

            @pl.when(c == 1)
            def _():
                pltpu.sync_copy(z_v.at[pl.ds(0, RRT)],
                                out_hbm.at[1, pl.ds(base + s * RRT, RRT)])

            plsc.subcore_barrier()

    mesh = plsc.VectorSubcoreMesh(
        core_axis_name="c", subcore_axis_name="s", num_cores=NC, num_subcores=NS
    )
    return pl.kernel(
        body,
        out_type=jax.ShapeDtypeStruct((NC, NP, HD), jnp.float32),
        mesh=mesh,
        scratch_types=[
            pltpu.VMEM((CH, HD), jnp.float32),
            pltpu.VMEM((CH,), jnp.int32),
            pltpu.VMEM((CH,), jnp.int32),
            pltpu.VMEM((ART, HD), jnp.float32),
            pltpu.VMEM_SHARED((AR, HD), jnp.float32),
        ],
    )


def _node_body(nf_ref, agga_ref, aggb_ref, u1a_ref, u1b0_ref, u1b1_ref,
               bu1_ref, u2_ref, bu2_ref, gamma_ref, beta_ref, out_ref):
    nf = nf_ref[...]
    a0 = agga_ref[0] + aggb_ref[0]
    a1 = agga_ref[1] + aggb_ref[1]
    t = (jnp.dot(nf, u1a_ref[...], preferred_element_type=jnp.float32)
         + jnp.dot(a0, u1b0_ref[...], preferred_element_type=jnp.float32)
         + jnp.dot(a1, u1b1_ref[...], preferred_element_type=jnp.float32)
         + bu1_ref[...])
    u = jnp.dot(jax.nn.relu(t), u2_ref[...], preferred_element_type=jnp.float32) + bu2_ref[...]
    res = nf + u
    mean = jnp.mean(res, axis=-1, keepdims=True)
    var = jnp.mean((res - mean) ** 2, axis=-1, keepdims=True)
    out_ref[...] = (res - mean) * jax.lax.rsqrt(var + 1e-5) * gamma_ref[...] + beta_ref[...]


def _node_mlp(nf, agga, aggb, U1, bu1, U2, bu2, gamma, beta):
    bn = 2000
    return pl.pallas_call(
        _node_body,
        grid=(N // bn,),
        in_specs=[
            pl.BlockSpec((bn, D), lambda i: (i, 0)),
            pl.BlockSpec((NC, bn, HD), lambda i: (0, i, 0)),
            pl.BlockSpec((NC, bn, HD), lambda i: (0, i, 0)),
            pl.BlockSpec((D, D), lambda i: (0, 0)),
            pl.BlockSpec((HD, D), lambda i: (0, 0)),
            pl.BlockSpec((HD, D), lambda i: (0, 0)),
            pl.BlockSpec((1, D), lambda i: (0, 0)),
            pl.BlockSpec((D, D), lambda i: (0, 0)),
            pl.BlockSpec((1, D), lambda i: (0, 0)),
            pl.BlockSpec((1, D), lambda i: (0, 0)),
            pl.BlockSpec((1, D), lambda i: (0, 0)),
        ],
        out_specs=pl.BlockSpec((bn, D), lambda i: (i, 0)),
        out_shape=jax.ShapeDtypeStruct((N, D), jnp.float32),
    )(nf, agga, aggb, U1[:D], U1[D : D + HD], U1[D + HD :], bu1.reshape(1, -1),
      U2, bu2.reshape(1, -1), gamma.reshape(1, -1), beta.reshape(1, -1))


def kernel(node_features, edge_features, edge_index, W1, b1, W2, b2,
           U1, bu1, U2, bu2, gamma, beta):
    src = edge_index[0]
    tgt = edge_index[1]
    Wcat = jnp.concatenate([W1[:D], W1[D : 2 * D]], axis=1)
    P1, P2 = _node_proj(node_features, Wcat)

    W1c = W1[2 * D :].astype(jnp.bfloat16)
    b1r = b1.reshape(1, -1)
    W2a = W2[:D].astype(jnp.bfloat16)
    W2b = W2[D:].astype(jnp.bfloat16)
    b2r = b2.reshape(1, -1)
    zeros_art = jnp.zeros((ART, HD), jnp.float32)

    g1a, g2a = _make_sc_gather(0, EA)(P1, P2, src, tgt)
    g1b, g2b = _make_sc_gather(EA, EB)(P1, P2, src, tgt)
    ma = _edge_mlp(0, EA, g1a, g2a, edge_features, W1c, b1r, W2a, W2b, b2r)
    mb = _edge_mlp(EA, EB, g1b, g2b, edge_features, W1c, b1r, W2a, W2b, b2r)
    agga = _make_sc_scatter(0, EA)(ma, tgt, zeros_art)
    aggb = _make_sc_scatter(EA, EB)(mb, tgt, zeros_art)
    return _node_mlp(node_features, agga, aggb, U1, bu1, U2, bu2, gamma, beta)


# speedup vs baseline: 2.7542x; 1.0545x over previous
import functools
import jax
import jax.numpy as jnp
from jax import lax
from jax.experimental import pallas as pl
from jax.experimental.pallas import tpu as pltpu
from jax.experimental.pallas import tpu_sc as plsc

N, E, D = 10000, 160000, 256
NC, NS = 2, 16
HD = D // 2
CH = 400
CHG = 200
NROUND = 3
RR = 3584
NP = NROUND * RR
AR = RR + 128
ART = AR // NS
RRT = RR // NS
EA, EB = 76800, 83200
BE = 1600


def _pack_bf16_pair(hi, lo):
    hb = jax.lax.bitcast_convert_type(hi.astype(jnp.bfloat16).astype(jnp.float32), jnp.int32)
    lb = jax.lax.bitcast_convert_type(lo.astype(jnp.bfloat16).astype(jnp.float32), jnp.int32)
    return hb | jax.lax.shift_right_logical(lb, 16)


def _unpack_hi(w):
    return jax.lax.bitcast_convert_type(w & jnp.int32(-65536), jnp.float32)


def _unpack_lo(w):
    return jax.lax.bitcast_convert_type(jax.lax.shift_left(w, 16), jnp.float32)


def _proj_body(nf_ref, w_ref, p1_ref, p2_ref):
    p = jnp.dot(nf_ref[...], w_ref[...], preferred_element_type=jnp.float32)
    p1 = p[:, : 2 * D]
    p2 = p[:, 2 * D :]
    p1_ref[...] = _pack_bf16_pair(p1[:, :D], p1[:, D:])
    p2_ref[...] = _pack_bf16_pair(p2[:, :D], p2[:, D:])


def _node_proj(nf, W12):
    bn = 2000
    return pl.pallas_call(
        _proj_body,
        grid=(N // bn,),
        in_specs=[
            pl.BlockSpec((bn, D), lambda i: (i, 0)),
            pl.BlockSpec((D, 4 * D), lambda i: (0, 0)),
        ],
        out_specs=[
            pl.BlockSpec((bn, D), lambda i: (i, 0)),
            pl.BlockSpec((bn, D), lambda i: (i, 0)),
        ],
        out_shape=[
            jax.ShapeDtypeStruct((N, D), jnp.int32),
            jax.ShapeDtypeStruct((N, D), jnp.int32),
        ],
    )(nf, W12)


def _make_sc_gather(e_off, e_len):
    eptg = e_len // NS
    nch = eptg // CHG

    def endpoint(tab, idx_hbm, out, idx_all, ra, rb, sa, sb, s):
        lb = s * eptg
        pltpu.sync_copy(idx_hbm.at[pl.ds(e_off + lb, eptg)], idx_all)

        def start(i, buf, sem):
            pltpu.async_copy(tab.at[idx_all.at[pl.ds(i * CHG, CHG)]], buf, sem)

        def wait(buf, sem):
            pltpu.make_async_copy(tab.at[pl.ds(0, CHG)], buf, sem).wait()

        start(0, ra, sa)

        def body(j, _):
            i0 = 2 * j
            start(i0 + 1, rb, sb)
            wait(ra, sa)
            pltpu.sync_copy(ra, out.at[pl.ds(lb + i0 * CHG, CHG)])

            @pl.when(j < nch // 2 - 1)
            def _():
                start(i0 + 2, ra, sa)

            wait(rb, sb)
            pltpu.sync_copy(rb, out.at[pl.ds(lb + (i0 + 1) * CHG, CHG)])
            return 0

        lax.fori_loop(0, nch // 2, body, 0)

    def body(p1_hbm, p2_hbm, src_hbm, tgt_hbm, g1_hbm, g2_hbm,
             idx_all, rows_a, rows_b, sem_a, sem_b):
        c = lax.axis_index("c")
        s = lax.axis_index("s")

        @pl.when(c == 0)
        def _():
            endpoint(p1_hbm, src_hbm, g1_hbm, idx_all, rows_a, rows_b,
                     sem_a, sem_b, s)

        @pl.when(c == 1)
        def _():
            endpoint(p2_hbm, tgt_hbm, g2_hbm, idx_all, rows_a, rows_b,
                     sem_a, sem_b, s)

    mesh = plsc.VectorSubcoreMesh(
        core_axis_name="c", subcore_axis_name="s", num_cores=NC, num_subcores=NS
    )
    return pl.kernel(
        body,
        out_type=[
            jax.ShapeDtypeStruct((e_len, D), jnp.int32),
            jax.ShapeDtypeStruct((e_len, D), jnp.int32),
        ],
        mesh=mesh,
        scratch_types=[
            pltpu.VMEM((eptg,), jnp.int32),
            pltpu.VMEM((CHG, D), jnp.int32),
            pltpu.VMEM((CHG, D), jnp.int32),
            pltpu.SemaphoreType.DMA,
            pltpu.SemaphoreType.DMA,
        ],
    )


def _edge_body(g1_ref, g2_ref, ef_ref, w1c_ref, b1_ref, w2a_ref, w2b_ref,
               b2_ref, m_ref):
    g1w = g1_ref[...]
    g2w = g2_ref[...]
    ef = ef_ref[...].astype(jnp.bfloat16)
    t = jnp.dot(ef, w1c_ref[...], preferred_element_type=jnp.float32) + b1_ref[...]
    h1 = jax.nn.relu(_unpack_hi(g1w) + _unpack_hi(g2w) + t[:, :D]).astype(jnp.bfloat16)
    h2 = jax.nn.relu(_unpack_lo(g1w) + _unpack_lo(g2w) + t[:, D:]).astype(jnp.bfloat16)
    m = (jnp.dot(h1, w2a_ref[...], preferred_element_type=jnp.float32)
         + jnp.dot(h2, w2b_ref[...], preferred_element_type=jnp.float32)
         + b2_ref[...])
    m_ref[0] = m[:, :HD]
    m_ref[1] = m[:, HD:]


def _edge_mlp(e_off, e_len, g1, g2, ef, W1c, b1r, W2a, W2b, b2r):
    ob = e_off // BE
    return pl.pallas_call(
        _edge_body,
        grid=(e_len // BE,),
        in_specs=[
            pl.BlockSpec((BE, D), lambda i: (i, 0)),
            pl.BlockSpec((BE, D), lambda i: (i, 0)),
            pl.BlockSpec((BE, D), lambda i: (i + ob, 0)),
            pl.BlockSpec((D, 2 * D), lambda i: (0, 0)),
            pl.BlockSpec((1, 2 * D), lambda i: (0, 0)),
            pl.BlockSpec((D, D), lambda i: (0, 0)),
            pl.BlockSpec((D, D), lambda i: (0, 0)),
            pl.BlockSpec((1, D), lambda i: (0, 0)),
        ],
        out_specs=pl.BlockSpec((NC, BE, HD), lambda i: (0, i, 0)),
        out_shape=jax.ShapeDtypeStruct((NC, e_len, HD), jnp.float32),
    )(g1, g2, ef, W1c, b1r, W2a, W2b, b2r)


def _make_sc_scatter(e_off, e_len):
    ept = e_len // NS
    nch = ept // CH

    def body(m_hbm, tgt_hbm, zeros_hbm, out_hbm, m_v, idx_v, idx2_v, z_v,
             acc_sh, sem_m):
        c = lax.axis_index("c")
        s = lax.axis_index("s")
        for r in range(NROUND):
            base = r * RR
            pltpu.sync_copy(zeros_hbm, z_v)
            pltpu.sync_copy(z_v, acc_sh.at[pl.ds(s * ART, ART)])
            plsc.subcore_barrier()

            def chunk(i, _):
                lb = s * ept + i * CH
                pltpu.async_copy(m_hbm.at[pl.ds(c * e_len + lb, CH)], m_v, sem_m)
                pltpu.sync_copy(tgt_hbm.at[pl.ds(e_off + lb, CH)], idx_v)
                for k in range(CH // 16):
                    v = idx_v[pl.ds(k * 16, 16)]
                    rel = v - base
                    inb = (rel >= 0) & (rel < RR)
                    trash = RR + (v & 127)
                    idx2_v[pl.ds(k * 16, 16)] = jnp.where(inb, rel, trash)
                pltpu.make_async_copy(m_hbm.at[pl.ds(0, CH)], m_v, sem_m).wait()
                pltpu.sync_copy(m_v, acc_sh.at[idx2_v], add=True)
                return 0

            lax.fori_loop(0, nch, chunk, 0)
            plsc.subcore_barrier()
            pltpu.sync_copy(acc_sh.at[pl.ds(s * RRT, RRT)], z_v.at[pl.ds(0, RRT)])

            @pl.when(c == 0)
            def _():
                pltpu.sync_copy(z_v.at[pl.ds(0, RRT)],
                                out_hbm.at[0, pl.ds(base + s * RRT, RRT)])

            @pl.when(c == 1)
            def _():
                pltpu.sync_copy(z_v.at[pl.ds(0, RRT)],
                                out_hbm.at[1, pl.ds(base + s * RRT, RRT)])

            plsc.subcore_barrier()

    mesh = plsc.VectorSubcoreMesh(
        core_axis_name="c", subcore_axis_name="s", num_cores=NC, num_subcores=NS
    )
    return pl.kernel(
        body,
        out_type=jax.ShapeDtypeStruct((NC, NP, HD), jnp.float32),
        mesh=mesh,
        scratch_types=[
            pltpu.VMEM((CH, HD), jnp.float32),
            pltpu.VMEM((CH,), jnp.int32),
            pltpu.VMEM((CH,), jnp.int32),
            pltpu.VMEM((ART, HD), jnp.float32),
            pltpu.VMEM_SHARED((AR, HD), jnp.float32),
            pltpu.SemaphoreType.DMA,
        ],
    )


def _node_body(nf_ref, agga_ref, aggb_ref, u1a_ref, u1b0_ref, u1b1_ref,
               bu1_ref, u2_ref, bu2_ref, gamma_ref, beta_ref, out_ref):
    nf = nf_ref[...]
    a0 = agga_ref[0] + aggb_ref[0]
    a1 = agga_ref[1] + aggb_ref[1]
    t = (jnp.dot(nf, u1a_ref[...], preferred_element_type=jnp.float32)
         + jnp.dot(a0, u1b0_ref[...], preferred_element_type=jnp.float32)
         + jnp.dot(a1, u1b1_ref[...], preferred_element_type=jnp.float32)
         + bu1_ref[...])
    u = jnp.dot(jax.nn.relu(t), u2_ref[...], preferred_element_type=jnp.float32) + bu2_ref[...]
    res = nf + u
    mean = jnp.mean(res, axis=-1, keepdims=True)
    var = jnp.mean((res - mean) ** 2, axis=-1, keepdims=True)
    out_ref[...] = (res - mean) * jax.lax.rsqrt(var + 1e-5) * gamma_ref[...] + beta_ref[...]


def _node_mlp(nf, agga, aggb, U1, bu1, U2, bu2, gamma, beta):
    bn = 2000
    return pl.pallas_call(
        _node_body,
        grid=(N // bn,),
        in_specs=[
            pl.BlockSpec((bn, D), lambda i: (i, 0)),
            pl.BlockSpec((NC, bn, HD), lambda i: (0, i, 0)),
            pl.BlockSpec((NC, bn, HD), lambda i: (0, i, 0)),
            pl.BlockSpec((D, D), lambda i: (0, 0)),
            pl.BlockSpec((HD, D), lambda i: (0, 0)),
            pl.BlockSpec((HD, D), lambda i: (0, 0)),
            pl.BlockSpec((1, D), lambda i: (0, 0)),
            pl.BlockSpec((D, D), lambda i: (0, 0)),
            pl.BlockSpec((1, D), lambda i: (0, 0)),
            pl.BlockSpec((1, D), lambda i: (0, 0)),
            pl.BlockSpec((1, D), lambda i: (0, 0)),
        ],
        out_specs=pl.BlockSpec((bn, D), lambda i: (i, 0)),
        out_shape=jax.ShapeDtypeStruct((N, D), jnp.float32),
    )(nf, agga, aggb, U1[:D], U1[D : D + HD], U1[D + HD :], bu1.reshape(1, -1),
      U2, bu2.reshape(1, -1), gamma.reshape(1, -1), beta.reshape(1, -1))


def kernel(node_features, edge_features, edge_index, W1, b1, W2, b2,
           U1, bu1, U2, bu2, gamma, beta):
    src = edge_index[0]
    tgt = edge_index[1]
    Wcat = jnp.concatenate([W1[:D], W1[D : 2 * D]], axis=1)
    P1, P2 = _node_proj(node_features, Wcat)

    W1c = W1[2 * D :].astype(jnp.bfloat16)
    b1r = b1.reshape(1, -1)
    W2a = W2[:D].astype(jnp.bfloat16)
    W2b = W2[D:].astype(jnp.bfloat16)
    b2r = b2.reshape(1, -1)
    zeros_art = jnp.zeros((ART, HD), jnp.float32)

    g1a, g2a = _make_sc_gather(0, EA)(P1, P2, src, tgt)
    g1b, g2b = _make_sc_gather(EA, EB)(P1, P2, src, tgt)
    ma = _edge_mlp(0, EA, g1a, g2a, edge_features, W1c, b1r, W2a, W2b, b2r)
    mb = _edge_mlp(EA, EB, g1b, g2b, edge_features, W1c, b1r, W2a, W2b, b2r)
    agga = _make_sc_scatter(0, EA)(ma.reshape(NC * EA, HD), tgt, zeros_art)
    aggb = _make_sc_scatter(EA, EB)(mb.reshape(NC * EB, HD), tgt, zeros_art)
    return _node_mlp(node_features, agga, aggb, U1, bu1, U2, bu2, gamma, beta)
